# baseline (speedup 1.0000x reference)
import functools

import jax
import jax.numpy as jnp
from jax import lax
from jax.experimental import pallas as pl
from jax.experimental.pallas import tpu as pltpu
from jax.experimental.pallas import tpu_sc as plsc

D_MODEL = 2048
D_MID = 4096
D_FEAT = 16384
B = 2048
K_MID = 128
K_FEAT = 32

_PREC = lax.Precision.DEFAULT

_NW = 32
_NB1 = 32768
_NB2 = 65536
_NBPOS = 16384
_SC_CHUNK = 16384
_SC_UNROLL = 8


def _sc_hist_body(h_hbm, pref_hbm, reg_hbm, out_hbm, pref_v, reg_v,
                  buf_a, buf_b, hist_v, sem_a, sem_b, *, mode, nseg, nbuck):
    c = lax.axis_index("c")
    s = lax.axis_index("s")
    wid = s * 2 + c
    base = wid * (_SC_CHUNK * nseg)

    zeros16 = jnp.zeros((16,), jnp.int32)

    @plsc.parallel_loop(0, nbuck, step=16, unroll=_SC_UNROLL)
    def _(j):
        hist_v[pl.ds(j, 16)] = zeros16

    pltpu.sync_copy(pref_hbm, pref_v)
    pltpu.sync_copy(reg_hbm, reg_v)
    pv = pref_v[...]
    rv = reg_v[...]
    ones16 = jnp.ones((16,), jnp.int32)
    iota16 = jnp.arange(16, dtype=jnp.int32)

    def process(buf, segbase):
        @plsc.parallel_loop(0, _SC_CHUNK, step=16, unroll=_SC_UNROLL)
        def _(i):
            v = buf[pl.ds(i, 16)]
            bits = plsc.bitcast(v, jnp.int32)
            if mode in (0, 1):
                mag = jnp.bitwise_and(bits, jnp.int32(0x7FFFFFFF))
                pos = bits > 0
                if mode == 1:
                    hi = jnp.right_shift(mag, 16)
                    msk = jnp.logical_and(pos, hi == pv)
                    idx = jnp.bitwise_and(mag, jnp.int32(0xFFFF))
                else:
                    msk = pos
                    idx = jnp.right_shift(mag, 16)
            else:
                gpos = segbase + i + iota16
                tie = bits == pv
                if mode == 3:
                    msk = jnp.logical_and(tie, jnp.right_shift(gpos, 14) == rv)
                    idx = jnp.bitwise_and(gpos, jnp.int32(_NBPOS - 1))
                else:
                    msk = tie
                    idx = jnp.right_shift(gpos, 14)
            plsc.addupdate_scatter(hist_v, [idx], ones16, mask=msk)

    def src(seg):
        return h_hbm.at[pl.ds(base + seg * _SC_CHUNK, _SC_CHUNK)]

    npair = nseg // 2
    pltpu.async_copy(src(0), buf_a, sem_a)

    def pair_body(p, carry):
        pltpu.async_copy(src(2 * p + 1), buf_b, sem_b)
        pltpu.make_async_copy(src(2 * p), buf_a, sem_a).wait()
        process(buf_a, base + (2 * p) * _SC_CHUNK)

        @pl.when(p < npair - 1)
        def _():
            pltpu.async_copy(src(2 * p + 2), buf_a, sem_a)

        pltpu.make_async_copy(src(2 * p + 1), buf_b, sem_b).wait()
        process(buf_b, base + (2 * p + 1) * _SC_CHUNK)
        return carry

    lax.fori_loop(0, npair, pair_body, 0)
    pltpu.sync_copy(hist_v, out_hbm.at[wid])


def _sc_hist(flat, pref, reg, mode, nbuck):
    n = flat.shape[0]
    nseg = n // (_NW * _SC_CHUNK)
    mesh = plsc.VectorSubcoreMesh(core_axis_name="c", subcore_axis_name="s")
    body = functools.partial(_sc_hist_body, mode=mode, nseg=nseg, nbuck=nbuck)
    k = pl.kernel(
        body,
        out_type=jax.ShapeDtypeStruct((_NW, nbuck), jnp.int32),
        mesh=mesh,
        compiler_params=pltpu.CompilerParams(needs_layout_passes=False),
        scratch_types=[
            pltpu.VMEM((16,), jnp.int32),
            pltpu.VMEM((16,), jnp.int32),
            pltpu.VMEM((_SC_CHUNK,), jnp.float32),
            pltpu.VMEM((_SC_CHUNK,), jnp.float32),
            pltpu.VMEM((nbuck,), jnp.int32),
            pltpu.SemaphoreType.DMA,
            pltpu.SemaphoreType.DMA,
        ],
    )
    return k(flat, jnp.full((16,), pref, jnp.int32),
             jnp.full((16,), reg, jnp.int32))


def _select_threshold_sc(z, target):
    flat = z.reshape(-1)
    n = flat.shape[0]
    zero = jnp.int32(0)

    hist1 = _sc_hist(flat, zero, zero, 0, _NB1)
    h1 = jnp.sum(hist1, axis=0, dtype=jnp.int32)
    s1 = jnp.cumsum(h1[::-1], dtype=jnp.int32)[::-1]
    s1e = jnp.concatenate([s1, jnp.zeros((1,), jnp.int32)])
    p_star = jnp.clip(jnp.sum((s1 >= target).astype(jnp.int32)) - 1,
                      0, _NB1 - 1)
    above = s1e[p_star + 1]
    t2 = target - above

    hist2 = _sc_hist(flat, p_star, zero, 1, _NB2)
    h2 = jnp.sum(hist2, axis=0, dtype=jnp.int32)
    s2 = jnp.cumsum(h2[::-1], dtype=jnp.int32)[::-1]
    s2e = jnp.concatenate([s2, jnp.zeros((1,), jnp.int32)])
    l_star = jnp.clip(jnp.sum((s2 >= t2).astype(jnp.int32)) - 1,
                      0, _NB2 - 1)
    cnt = above + s2[l_star]
    count_gt = above + s2e[l_star + 1]
    t_bits = jnp.bitwise_or(jnp.left_shift(p_star, 16), l_star)
    t = lax.bitcast_convert_type(t_bits, jnp.float32)

    j = target - count_gt
    nba = n >> 14
    hist_a = _sc_hist(flat, t_bits, zero, 2, nba)
    ha = jnp.sum(hist_a, axis=0, dtype=jnp.int32)
    ca = jnp.cumsum(ha, dtype=jnp.int32)
    total_ties = ca[nba - 1]
    j_ex = total_ties - j
    rstar = jnp.clip(jnp.sum((ca < j_ex).astype(jnp.int32)), 0, nba - 1)
    before = jnp.where(rstar > 0, ca[jnp.clip(rstar - 1, 0, nba - 1)], 0)
    j2 = j_ex - before

    hist_b = _sc_hist(flat, t_bits, rstar, 3, _NBPOS)
    hb = jnp.sum(hist_b, axis=0, dtype=jnp.int32)
    cb = jnp.cumsum(hb, dtype=jnp.int32)
    off = jnp.clip(jnp.sum((cb < j2).astype(jnp.int32)), 0, _NBPOS - 1)
    idx_cut = rstar * _NBPOS + off
    idx_cut = jnp.where(j_ex <= 0, jnp.int32(-1), idx_cut)
    return t, jnp.minimum(cnt, target), idx_cut


def _mask(a, t_ref, ic_ref, m, k, bm, bk, kdim):
    t = t_ref[0, 0]
    rows = m * bm + lax.broadcasted_iota(jnp.int32, a.shape, 0)
    cols = k * bk + lax.broadcasted_iota(jnp.int32, a.shape, 1)
    flat = rows * kdim + cols
    keep = (a > t) | ((a == t) & (flat > ic_ref[0, 0]))
    return jnp.where(keep, a, 0.0)


def _mm_body(t_ref, ic_ref, a_ref, b_ref, bias_ref, o_ref, *,
             masked, nk, bm, bk, kdim):
    m, k = pl.program_id(0), pl.program_id(2)
    a = a_ref[...]
    if masked:
        a = _mask(a, t_ref, ic_ref, m, k, bm, bk, kdim)
    d = jnp.dot(a, b_ref[...], preferred_element_type=jnp.float32,
                precision=_PREC)
    if nk == 1:
        o_ref[...] = d + bias_ref[...]
    else:
        @pl.when(k == 0)
        def _():
            o_ref[...] = d

        @pl.when((k > 0) & (k < nk - 1))
        def _():
            o_ref[...] += d

        @pl.when(k == nk - 1)
        def _():
            o_ref[...] = o_ref[...] + d + bias_ref[...]


def _matmul(a, b, bias, t, ic, bm, bn, bk):
    M, K = a.shape
    _, N = b.shape
    nm, nn, nk = M // bm, N // bn, K // bk
    masked = t is not None
    tt = t.reshape(1, 1) if masked else jnp.zeros((1, 1), jnp.float32)
    ii = ic.reshape(1, 1) if masked else jnp.zeros((1, 1), jnp.int32)
    body = functools.partial(_mm_body, masked=masked, nk=nk, bm=bm, bk=bk,
                             kdim=K)
    return pl.pallas_call(
        body,
        grid=(nm, nn, nk),
        in_specs=[
            pl.BlockSpec(memory_space=pltpu.SMEM),
            pl.BlockSpec(memory_space=pltpu.SMEM),
            pl.BlockSpec((bm, bk), lambda m, n, k: (m, k)),
            pl.BlockSpec((bk, bn), lambda m, n, k: (k, n)),
            pl.BlockSpec((1, bn), lambda m, n, k: (0, n)),
        ],
        out_specs=pl.BlockSpec((bm, bn), lambda m, n, k: (m, n)),
        out_shape=jax.ShapeDtypeStruct((M, N), jnp.float32),
    )(tt, ii, a, b, bias.reshape(1, -1))


def _dec1_body(t_ref, ic_ref, a_ref, b_ref, bias_ref, x_ref, o_ref, l2_ref,
               acc_ref, *, nk, bm, bk, kdim):
    m, n, k = pl.program_id(0), pl.program_id(1), pl.program_id(2)

    @pl.when((m == 0) & (n == 0) & (k == 0))
    def _():
        l2_ref[...] = jnp.zeros_like(l2_ref)

    @pl.when(k == 0)
    def _():
        acc_ref[...] = jnp.zeros_like(acc_ref)

    a = _mask(a_ref[...], t_ref, ic_ref, m, k, bm, bk, kdim)
    acc_ref[...] += jnp.dot(a, b_ref[...], preferred_element_type=jnp.float32,
                            precision=_PREC)

    @pl.when(k == nk - 1)
    def _():
        res = acc_ref[...] + bias_ref[...]
        o_ref[...] = res
        d = res - x_ref[...]
        s = jnp.sum(d * d)
        l2_ref[...] = l2_ref[...] + jnp.full((1, 128), s / 128.0, jnp.float32)


def _dec1_matmul(a, b, bias, t, ic, x, bm, bn, bk):
    M, K = a.shape
    _, N = b.shape
    nm, nn, nk = M // bm, N // bn, K // bk
    return pl.pallas_call(
        functools.partial(_dec1_body, nk=nk, bm=bm, bk=bk, kdim=K),
        grid=(nm, nn, nk),
        in_specs=[
            pl.BlockSpec(memory_space=pltpu.SMEM),
            pl.BlockSpec(memory_space=pltpu.SMEM),
            pl.BlockSpec((bm, bk), lambda m, n, k: (m, k)),
            pl.BlockSpec((bk, bn), lambda m, n, k: (k, n)),
            pl.BlockSpec((1, bn), lambda m, n, k: (0, n)),
            pl.BlockSpec((bm, bn), lambda m, n, k: (m, n)),
        ],
        out_specs=[
            pl.BlockSpec((bm, bn), lambda m, n, k: (m, n)),
            pl.BlockSpec((1, 128), lambda m, n, k: (0, 0)),
        ],
        out_shape=[
            jax.ShapeDtypeStruct((M, N), jnp.float32),
            jax.ShapeDtypeStruct((1, 128), jnp.float32),
        ],
        scratch_shapes=[pltpu.VMEM((bm, bn), jnp.float32)],
    )(t.reshape(1, 1), ic.reshape(1, 1), a, b, bias.reshape(1, -1), x)


def kernel(x, W_enc1, b_enc1, W_enc2, b_enc2, W_dec2, b_dec2, W_dec1, b_dec1):
    z1 = _matmul(x, W_enc1, b_enc1, None, None, bm=1024, bn=1024, bk=1024)
    t1, _, ic1 = _select_threshold_sc(z1, K_MID * B)

    z2 = _matmul(z1, W_enc2, b_enc2, t1, ic1, bm=2048, bn=1024, bk=1024)
    t2, cnt2, ic2 = _select_threshold_sc(z2, K_FEAT * B)

    z3 = _matmul(z2, W_dec2, b_dec2, t2, ic2, bm=2048, bn=1024, bk=1024)
    t3, _, ic3 = _select_threshold_sc(z3, K_MID * B)

    recon, l2part = _dec1_matmul(z3, W_dec1, b_dec1, t3, ic3, x,
                                 bm=1024, bn=1024, bk=1024)

    l2_loss = jnp.sum(l2part) / (B * D_MODEL)
    l0_norm = cnt2.astype(jnp.float32) / B
    n_dead = jnp.zeros((D_FEAT,), dtype=bool)
    return recon, l2_loss, l0_norm, n_dead
